# baseline (speedup 1.0000x reference)
import functools
import math

import jax
import jax.numpy as jnp
from jax import lax
from jax.experimental import pallas as pl
from jax.experimental.pallas import tpu as pltpu
from jax.experimental.pallas import tpu_sc as plsc

_NUM_LEVELS = 16
_LEVEL_DIM = 2
_BASE_RES = 16
_MAX_RES = 2048
_LOG2_HASH = 19
_HASH = 2 ** _LOG2_HASH
_SCALE = math.exp((math.log(_MAX_RES) - math.log(_BASE_RES)) / (_NUM_LEVELS - 1))
_RES = [int(_BASE_RES * _SCALE ** l) for l in range(_NUM_LEVELS)]
_R2M = [(r * r) % _HASH for r in _RES]

_NC = 2
_NS = 16
_NW = _NC * _NS

_P = 1024
_W = _NUM_LEVELS * _LEVEL_DIM


def _sc_body(x0, x1, x2, tabw, resf_h, resi_h, r2m_h, out,
             c0_v, c1_v, c2_v, resf_v, resi_v, r2m_v, fidx_v, gat_v, sem,
             *, n_points):
    ppw = n_points // _NW
    nchunk = ppw // _P
    wid = lax.axis_index("s") * _NC + lax.axis_index("c")
    base = wid * ppw

    pltpu.sync_copy(resf_h, resf_v)
    pltpu.sync_copy(resi_h, resi_v)
    pltpu.sync_copy(r2m_h, r2m_v)

    for chunk in range(nchunk):
        cb = base + chunk * _P
        pltpu.sync_copy(x0.at[pl.ds(cb, _P)], c0_v)
        pltpu.sync_copy(x1.at[pl.ds(cb, _P)], c1_v)
        pltpu.sync_copy(x2.at[pl.ds(cb, _P)], c2_v)

        def grp_body(g, _):
            resf = resf_v[...]
            resi = resi_v[...]
            r2m = r2m_v[...]
            lane = lax.iota(jnp.int32, 16)
            lvl = lane << _LOG2_HASH
            half = lane >> 1
            parity = lane & 1
            s = pl.ds(g * 16, 16)
            a0 = c0_v[s]
            a1 = c1_v[s]
            a2 = c2_v[s]
            for j in range(16):
                i0 = (resf * a0[j]).astype(jnp.int32)
                i1 = (resf * a1[j]).astype(jnp.int32)
                i2 = (resf * a2[j]).astype(jnp.int32)
                i0 = jnp.where(i0 >= resi, i0 - resi, i0)
                i1 = jnp.where(i1 >= resi, i1 - resi, i1)
                i2 = jnp.where(i2 >= resi, i2 - resi, i2)
                h = i0 + i1 * resi + i2 * r2m
                fidx = (h & (_HASH - 1)) | lvl
                lo = (fidx.at[half].get(mode="promise_in_bounds") << 1) + parity
                hi = (fidx.at[half + 8].get(mode="promise_in_bounds") << 1
                      ) + parity
                p = g * 16 + j
                fidx_v[pl.ds(p * _W, 16)] = lo
                fidx_v[pl.ds(p * _W + 16, 16)] = hi
            return 0

        lax.fori_loop(0, _P // 16, grp_body, 0)

        pltpu.async_copy(tabw.at[fidx_v], gat_v, sem).wait()
        pltpu.sync_copy(gat_v, out.at[pl.ds(cb * _W, _P * _W)])


def kernel(x, tables):
    b0, b1, _ = x.shape
    n = b0 * b1
    xt = x.reshape(n, 3).T
    tabw = tables.reshape(-1)
    resf = jnp.array(_RES, dtype=jnp.float32)
    resi = jnp.array(_RES, dtype=jnp.int32)
    r2m = jnp.array(_R2M, dtype=jnp.int32)

    mesh = plsc.VectorSubcoreMesh(core_axis_name="c", subcore_axis_name="s")
    run = pl.kernel(
        functools.partial(_sc_body, n_points=n),
        out_type=jax.ShapeDtypeStruct((n * _W,), jnp.float32),
        mesh=mesh,
        compiler_params=pltpu.CompilerParams(use_tc_tiling_on_sc=False),
        scratch_types=[
            pltpu.VMEM((_P,), jnp.float32),
            pltpu.VMEM((_P,), jnp.float32),
            pltpu.VMEM((_P,), jnp.float32),
            pltpu.VMEM((_NUM_LEVELS,), jnp.float32),
            pltpu.VMEM((_NUM_LEVELS,), jnp.int32),
            pltpu.VMEM((_NUM_LEVELS,), jnp.int32),
            pltpu.VMEM((_P * _W,), jnp.int32),
            pltpu.VMEM((_P * _W,), jnp.float32),
            pltpu.SemaphoreType.DMA,
        ],
    )
    out = run(xt[0], xt[1], xt[2], tabw, resf, resi, r2m)
    return out.reshape(b0, b1, _W)
